# Initial kernel scaffold; baseline (speedup 1.0000x reference)
import functools

import jax
import jax.numpy as jnp
from jax import lax
from jax.experimental import pallas as pl
from jax.experimental.pallas import tpu as pltpu
from jax.experimental.pallas import tpu_sc as plsc

N = 10000
D = 128
HD = 128
C = 40
E = 320000

NC = 2
NS = 16
NW = NC * NS
CHUNK = 128
NPAD = 10016
ROWS_PER_SUB = NPAD // NS
EDGES_PER_TILE = -(-E // (NW * CHUNK)) * CHUNK
EPAD = EDGES_PER_TILE * NW
NCHUNKS = EDGES_PER_TILE // CHUNK

BT = 512
_GRID = -(-N // BT)

_MESH = plsc.VectorSubcoreMesh(core_axis_name="c", subcore_axis_name="s")


def _const_fill(ref, nrows, ncols, value):
    @pl.loop(0, nrows)
    def _(r):
        @pl.loop(0, ncols, step=16)
        def _(c):
            ref[r, pl.ds(c, 16)] = jnp.full((16,), value, jnp.float32)


def _zero_shared_slab(zero_v, acc_sh, base):
    @pl.loop(0, 4)
    def _(i):
        pltpu.sync_copy(zero_v.at[pl.ds(0, 128)],
                        acc_sh.at[pl.ds(base + i * 128, 128)])
    pltpu.sync_copy(zero_v.at[pl.ds(0, 114)],
                    acc_sh.at[pl.ds(base + 4 * 128, 114)])


def _drain_shared_slab(acc_sh, out_core, base):
    @pl.loop(0, 4)
    def _(i):
        pltpu.sync_copy(acc_sh.at[pl.ds(base + i * 128, 128)],
                        out_core.at[pl.ds(base + i * 128, 128)])
    pltpu.sync_copy(acc_sh.at[pl.ds(base + 4 * 128, 114)],
                    out_core.at[pl.ds(base + 4 * 128, 114)])


@functools.partial(
    pl.kernel,
    out_type=jax.ShapeDtypeStruct((NC, NPAD, 16), jnp.float32),
    mesh=_MESH,
    scratch_types=[
        pltpu.VMEM((CHUNK,), jnp.int32),
        pltpu.VMEM((CHUNK, 16), jnp.float32),
        pltpu.VMEM((CHUNK, 16), jnp.float32),
        pltpu.VMEM_SHARED((NPAD, 16), jnp.float32),
    ],
)
def _sc_degree(dst_hbm, out_hbm, idx_v, ones_v, zero_v, acc_sh):
    cid = lax.axis_index("c")
    sid = lax.axis_index("s")
    wid = sid * NC + cid
    _const_fill(ones_v, CHUNK, 16, 1.0)
    _const_fill(zero_v, CHUNK, 16, 0.0)
    base = sid * ROWS_PER_SUB
    @pl.loop(0, 4)
    def _(i):
        pltpu.sync_copy(zero_v.at[pl.ds(0, 128)],
                        acc_sh.at[pl.ds(base + i * 128, 128)])
    pltpu.sync_copy(zero_v.at[pl.ds(0, 114)],
                    acc_sh.at[pl.ds(base + 4 * 128, 114)])
    plsc.subcore_barrier()

    e0 = wid * EDGES_PER_TILE
    @pl.loop(0, NCHUNKS)
    def _(i):
        pltpu.sync_copy(dst_hbm.at[pl.ds(e0 + i * CHUNK, CHUNK)], idx_v)
        pltpu.sync_copy(ones_v, acc_sh.at[idx_v], add=True)
    plsc.subcore_barrier()

    _drain_shared_slab(acc_sh, out_hbm.at[cid], base)


@functools.partial(
    pl.kernel,
    out_type=jax.ShapeDtypeStruct((NC, NPAD, HD), jnp.float32),
    mesh=_MESH,
    scratch_types=[
        pltpu.VMEM((CHUNK,), jnp.int32),
        pltpu.VMEM((CHUNK,), jnp.int32),
        pltpu.VMEM((CHUNK, HD), jnp.float32),
        pltpu.VMEM((CHUNK, HD), jnp.float32),
        pltpu.VMEM_SHARED((NPAD, HD), jnp.float32),
        pltpu.SemaphoreType.DMA,
    ],
)
def _sc_aggregate(g_hbm, src_hbm, dst_hbm, out_hbm,
                  sidx_v, didx_v, rows_v, zero_v, acc_sh, sem):
    cid = lax.axis_index("c")
    sid = lax.axis_index("s")
    wid = sid * NC + cid
    _const_fill(zero_v, CHUNK, HD, 0.0)
    base = sid * ROWS_PER_SUB
    _zero_shared_slab(zero_v, acc_sh, base)
    plsc.subcore_barrier()

    e0 = wid * EDGES_PER_TILE
    @pl.loop(0, NCHUNKS)
    def _(i):
        pltpu.sync_copy(src_hbm.at[pl.ds(e0 + i * CHUNK, CHUNK)], sidx_v)
        pltpu.sync_copy(dst_hbm.at[pl.ds(e0 + i * CHUNK, CHUNK)], didx_v)
        pltpu.async_copy(g_hbm.at[sidx_v], rows_v, sem).wait()
        pltpu.sync_copy(rows_v, acc_sh.at[didx_v], add=True)
    plsc.subcore_barrier()

    _drain_shared_slab(acc_sh, out_hbm.at[cid], base)


def _tc_matmul(x, w):
    def body(x_ref, w_ref, o_ref):
        o_ref[...] = jnp.dot(x_ref[...], w_ref[...],
                             preferred_element_type=jnp.float32)
    return pl.pallas_call(
        body,
        grid=(_GRID,),
        in_specs=[pl.BlockSpec((BT, x.shape[1]), lambda i: (i, 0)),
                  pl.BlockSpec(w.shape, lambda i: (0, 0))],
        out_specs=pl.BlockSpec((BT, w.shape[1]), lambda i: (i, 0)),
        out_shape=jax.ShapeDtypeStruct((N, w.shape[1]), jnp.float32),
    )(x, w)


def _tc_scale(h, deg0, deg1):
    def body(h_ref, d0_ref, d1_ref, g_ref, dv_ref):
        deg = d0_ref[...] + d1_ref[...] + 1.0
        dinv = lax.rsqrt(deg)
        g_ref[...] = h_ref[...] * dinv[:, :1]
        dv_ref[...] = dinv
    return pl.pallas_call(
        body,
        grid=(_GRID,),
        in_specs=[pl.BlockSpec((BT, HD), lambda i: (i, 0)),
                  pl.BlockSpec((BT, 16), lambda i: (i, 0)),
                  pl.BlockSpec((BT, 16), lambda i: (i, 0))],
        out_specs=[pl.BlockSpec((BT, HD), lambda i: (i, 0)),
                   pl.BlockSpec((BT, 16), lambda i: (i, 0))],
        out_shape=[jax.ShapeDtypeStruct((N, HD), jnp.float32),
                   jax.ShapeDtypeStruct((N, 16), jnp.float32)],
    )(h, deg0, deg1)


def _tc_combine_matmul(p0, p1, h, dv, b, w):
    def body(p0_ref, p1_ref, h_ref, dv_ref, b_ref, w_ref, hn_ref, gn_ref):
        dinv = dv_ref[...][:, :1]
        pre = (dinv * (p0_ref[...] + p1_ref[...])
               + (dinv * dinv) * h_ref[...] + b_ref[...])
        a = jnp.maximum(pre, 0.0)
        hn = jnp.dot(a, w_ref[...], preferred_element_type=jnp.float32)
        hn_ref[...] = hn
        gn_ref[...] = hn * dinv
    return pl.pallas_call(
        body,
        grid=(_GRID,),
        in_specs=[pl.BlockSpec((BT, HD), lambda i: (i, 0)),
                  pl.BlockSpec((BT, HD), lambda i: (i, 0)),
                  pl.BlockSpec((BT, HD), lambda i: (i, 0)),
                  pl.BlockSpec((BT, 16), lambda i: (i, 0)),
                  pl.BlockSpec((HD,), lambda i: (0,)),
                  pl.BlockSpec((HD, HD), lambda i: (0, 0))],
        out_specs=[pl.BlockSpec((BT, HD), lambda i: (i, 0)),
                   pl.BlockSpec((BT, HD), lambda i: (i, 0))],
        out_shape=[jax.ShapeDtypeStruct((N, HD), jnp.float32),
                   jax.ShapeDtypeStruct((N, HD), jnp.float32)],
    )(p0, p1, h, dv, b, w)


def _tc_head(p0, p1, h, dv, b, wout, bout):
    def body(p0_ref, p1_ref, h_ref, dv_ref, b_ref, w_ref, bo_ref, o_ref):
        dinv = dv_ref[...][:, :1]
        pre = (dinv * (p0_ref[...] + p1_ref[...])
               + (dinv * dinv) * h_ref[...] + b_ref[...])
        a = jnp.maximum(pre, 0.0)
        logits = jnp.dot(a, w_ref[...],
                         preferred_element_type=jnp.float32) + bo_ref[...]
        m = jnp.max(logits, axis=1, keepdims=True)
        ex = jnp.exp(logits - m)
        o_ref[...] = ex / jnp.sum(ex, axis=1, keepdims=True)
    return pl.pallas_call(
        body,
        grid=(_GRID,),
        in_specs=[pl.BlockSpec((BT, HD), lambda i: (i, 0)),
                  pl.BlockSpec((BT, HD), lambda i: (i, 0)),
                  pl.BlockSpec((BT, HD), lambda i: (i, 0)),
                  pl.BlockSpec((BT, 16), lambda i: (i, 0)),
                  pl.BlockSpec((HD,), lambda i: (0,)),
                  pl.BlockSpec((HD, C), lambda i: (0, 0)),
                  pl.BlockSpec((C,), lambda i: (0,))],
        out_specs=pl.BlockSpec((BT, C), lambda i: (i, 0)),
        out_shape=jax.ShapeDtypeStruct((N, C), jnp.float32),
    )(p0, p1, h, dv, b, wout, bout)


def kernel(X, edges_index, W1, b1, W2, b2, W3, b3, Wout, bout):
    src = edges_index[0].astype(jnp.int32)
    dst = edges_index[1].astype(jnp.int32)
    src = jnp.concatenate([src, jnp.zeros((EPAD - E,), jnp.int32)])
    dst = jnp.concatenate([dst, jnp.full((EPAD - E,), N, jnp.int32)])

    degp = _sc_degree(dst)
    h1 = _tc_matmul(X, W1)
    g1, dv = _tc_scale(h1, degp[0, :N], degp[1, :N])

    p = _sc_aggregate(g1, src, dst)
    h2, g2 = _tc_combine_matmul(p[0, :N], p[1, :N], h1, dv, b1, W2)
    p = _sc_aggregate(g2, src, dst)
    h3, g3 = _tc_combine_matmul(p[0, :N], p[1, :N], h2, dv, b2, W3)
    p = _sc_aggregate(g3, src, dst)
    return _tc_head(p[0, :N], p[1, :N], h3, dv, b3, Wout, bout)


# speedup vs baseline: 8.7613x; 8.7613x over previous
import functools

import jax
import jax.numpy as jnp
from jax import lax
from jax.experimental import pallas as pl
from jax.experimental.pallas import tpu as pltpu
from jax.experimental.pallas import tpu_sc as plsc

N = 10000
D = 128
HD = 128
C = 40
E = 320000

NC = 2
NS = 16
NW = NC * NS
CHUNK = 128
NPAD = 10112
ROWS_PER_SUB = NPAD // NS
EDGES_PER_TILE = -(-E // (NW * CHUNK)) * CHUNK
EPAD = EDGES_PER_TILE * NW
NCHUNKS = EDGES_PER_TILE // CHUNK

BT = 512
_GRID = -(-N // BT)

@functools.cache
def _sc_mesh():
    return plsc.VectorSubcoreMesh(core_axis_name="c", subcore_axis_name="s")


def _const_fill(ref, nrows, ncols, value):
    @pl.loop(0, nrows)
    def _(r):
        @pl.loop(0, ncols, step=16)
        def _(c):
            ref[r, pl.ds(c, 16)] = jnp.full((16,), value, jnp.float32)


def _zero_shared_slab(zero_v, acc_sh, base):
    @pl.loop(0, 4)
    def _(i):
        pltpu.sync_copy(zero_v.at[pl.ds(0, 128)],
                        acc_sh.at[pl.ds(base + i * 128, 128)])
    pltpu.sync_copy(zero_v.at[pl.ds(0, 120)],
                    acc_sh.at[pl.ds(base + 4 * 128, 120)])


def _drain_shared_slab(acc_sh, out_core, base):
    @pl.loop(0, 4)
    def _(i):
        pltpu.sync_copy(acc_sh.at[pl.ds(base + i * 128, 128)],
                        out_core.at[pl.ds(base + i * 128, 128)])
    pltpu.sync_copy(acc_sh.at[pl.ds(base + 4 * 128, 120)],
                    out_core.at[pl.ds(base + 4 * 128, 120)])


@functools.cache
def _sc_degree_kernel():
    return pl.kernel(
        _sc_degree_body,
        out_type=jax.ShapeDtypeStruct((NC, NPAD, 16), jnp.float32),
        mesh=_sc_mesh(),
        scratch_types=[
            pltpu.VMEM((CHUNK,), jnp.int32),
            pltpu.VMEM((CHUNK, 16), jnp.float32),
            pltpu.VMEM((CHUNK, 16), jnp.float32),
            pltpu.VMEM_SHARED((NPAD, 16), jnp.float32),
        ],
    )


def _sc_degree(dst):
    return _sc_degree_kernel()(dst)


def _sc_degree_body(dst_hbm, out_hbm, idx_v, ones_v, zero_v, acc_sh):
    cid = lax.axis_index("c")
    sid = lax.axis_index("s")
    wid = sid * NC + cid
    _const_fill(ones_v, CHUNK, 16, 1.0)
    _const_fill(zero_v, CHUNK, 16, 0.0)
    base = sid * ROWS_PER_SUB
    @pl.loop(0, 4)
    def _(i):
        pltpu.sync_copy(zero_v.at[pl.ds(0, 128)],
                        acc_sh.at[pl.ds(base + i * 128, 128)])
    pltpu.sync_copy(zero_v.at[pl.ds(0, 120)],
                    acc_sh.at[pl.ds(base + 4 * 128, 120)])
    plsc.subcore_barrier()

    e0 = wid * EDGES_PER_TILE
    @pl.loop(0, NCHUNKS)
    def _(i):
        pltpu.sync_copy(dst_hbm.at[pl.ds(e0 + i * CHUNK, CHUNK)], idx_v)
        pltpu.sync_copy(ones_v, acc_sh.at[idx_v], add=True)
    plsc.subcore_barrier()

    _drain_shared_slab(acc_sh, out_hbm.at[cid], base)


@functools.cache
def _sc_aggregate_kernel():
    return pl.kernel(
        _sc_aggregate_body,
        out_type=jax.ShapeDtypeStruct((NC, NPAD, HD), jnp.float32),
        mesh=_sc_mesh(),
        scratch_types=[
            pltpu.VMEM((CHUNK,), jnp.int32),
            pltpu.VMEM((CHUNK,), jnp.int32),
            pltpu.VMEM((CHUNK, HD), jnp.float32),
            pltpu.VMEM((CHUNK, HD), jnp.float32),
            pltpu.VMEM_SHARED((NPAD, HD), jnp.float32),
            pltpu.SemaphoreType.DMA,
        ],
    )


def _sc_aggregate(g, src, dst):
    return _sc_aggregate_kernel()(g, src, dst)


def _sc_aggregate_body(g_hbm, src_hbm, dst_hbm, out_hbm,
                       sidx_v, didx_v, rows_v, zero_v, acc_sh, sem):
    cid = lax.axis_index("c")
    sid = lax.axis_index("s")
    wid = sid * NC + cid
    _const_fill(zero_v, CHUNK, HD, 0.0)
    base = sid * ROWS_PER_SUB
    _zero_shared_slab(zero_v, acc_sh, base)
    plsc.subcore_barrier()

    e0 = wid * EDGES_PER_TILE
    @pl.loop(0, NCHUNKS)
    def _(i):
        pltpu.sync_copy(src_hbm.at[pl.ds(e0 + i * CHUNK, CHUNK)], sidx_v)
        pltpu.sync_copy(dst_hbm.at[pl.ds(e0 + i * CHUNK, CHUNK)], didx_v)
        pltpu.async_copy(g_hbm.at[sidx_v], rows_v, sem).wait()
        pltpu.sync_copy(rows_v, acc_sh.at[didx_v], add=True)
    plsc.subcore_barrier()

    _drain_shared_slab(acc_sh, out_hbm.at[cid], base)


def _tc_matmul(x, w):
    def body(x_ref, w_ref, o_ref):
        o_ref[...] = jnp.dot(x_ref[...], w_ref[...],
                             preferred_element_type=jnp.float32)
    return pl.pallas_call(
        body,
        grid=(_GRID,),
        in_specs=[pl.BlockSpec((BT, x.shape[1]), lambda i: (i, 0)),
                  pl.BlockSpec(w.shape, lambda i: (0, 0))],
        out_specs=pl.BlockSpec((BT, w.shape[1]), lambda i: (i, 0)),
        out_shape=jax.ShapeDtypeStruct((N, w.shape[1]), jnp.float32),
    )(x, w)


def _tc_scale(h, deg0, deg1):
    def body(h_ref, d0_ref, d1_ref, g_ref, dv_ref):
        deg = d0_ref[...] + d1_ref[...] + 1.0
        dinv = lax.rsqrt(deg)
        g_ref[...] = h_ref[...] * dinv[:, :1]
        dv_ref[...] = dinv
    return pl.pallas_call(
        body,
        grid=(_GRID,),
        in_specs=[pl.BlockSpec((BT, HD), lambda i: (i, 0)),
                  pl.BlockSpec((BT, 16), lambda i: (i, 0)),
                  pl.BlockSpec((BT, 16), lambda i: (i, 0))],
        out_specs=[pl.BlockSpec((BT, HD), lambda i: (i, 0)),
                   pl.BlockSpec((BT, 16), lambda i: (i, 0))],
        out_shape=[jax.ShapeDtypeStruct((N, HD), jnp.float32),
                   jax.ShapeDtypeStruct((N, 16), jnp.float32)],
    )(h, deg0, deg1)


def _tc_combine_matmul(p0, p1, h, dv, b, w):
    def body(p0_ref, p1_ref, h_ref, dv_ref, b_ref, w_ref, hn_ref, gn_ref):
        dinv = dv_ref[...][:, :1]
        pre = (dinv * (p0_ref[...] + p1_ref[...])
               + (dinv * dinv) * h_ref[...] + b_ref[...])
        a = jnp.maximum(pre, 0.0)
        hn = jnp.dot(a, w_ref[...], preferred_element_type=jnp.float32)
        hn_ref[...] = hn
        gn_ref[...] = hn * dinv
    return pl.pallas_call(
        body,
        grid=(_GRID,),
        in_specs=[pl.BlockSpec((BT, HD), lambda i: (i, 0)),
                  pl.BlockSpec((BT, HD), lambda i: (i, 0)),
                  pl.BlockSpec((BT, HD), lambda i: (i, 0)),
                  pl.BlockSpec((BT, 16), lambda i: (i, 0)),
                  pl.BlockSpec((HD,), lambda i: (0,)),
                  pl.BlockSpec((HD, HD), lambda i: (0, 0))],
        out_specs=[pl.BlockSpec((BT, HD), lambda i: (i, 0)),
                   pl.BlockSpec((BT, HD), lambda i: (i, 0))],
        out_shape=[jax.ShapeDtypeStruct((N, HD), jnp.float32),
                   jax.ShapeDtypeStruct((N, HD), jnp.float32)],
    )(p0, p1, h, dv, b, w)


def _tc_head(p0, p1, h, dv, b, wout, bout):
    def body(p0_ref, p1_ref, h_ref, dv_ref, b_ref, w_ref, bo_ref, o_ref):
        dinv = dv_ref[...][:, :1]
        pre = (dinv * (p0_ref[...] + p1_ref[...])
               + (dinv * dinv) * h_ref[...] + b_ref[...])
        a = jnp.maximum(pre, 0.0)
        logits = jnp.dot(a, w_ref[...],
                         preferred_element_type=jnp.float32) + bo_ref[...]
        m = jnp.max(logits, axis=1, keepdims=True)
        ex = jnp.exp(logits - m)
        o_ref[...] = ex / jnp.sum(ex, axis=1, keepdims=True)
    return pl.pallas_call(
        body,
        grid=(_GRID,),
        in_specs=[pl.BlockSpec((BT, HD), lambda i: (i, 0)),
                  pl.BlockSpec((BT, HD), lambda i: (i, 0)),
                  pl.BlockSpec((BT, HD), lambda i: (i, 0)),
                  pl.BlockSpec((BT, 16), lambda i: (i, 0)),
                  pl.BlockSpec((HD,), lambda i: (0,)),
                  pl.BlockSpec((HD, C), lambda i: (0, 0)),
                  pl.BlockSpec((C,), lambda i: (0,))],
        out_specs=pl.BlockSpec((BT, C), lambda i: (i, 0)),
        out_shape=jax.ShapeDtypeStruct((N, C), jnp.float32),
    )(p0, p1, h, dv, b, wout, bout)


def kernel(X, edges_index, W1, b1, W2, b2, W3, b3, Wout, bout):
    src = edges_index[0].astype(jnp.int32)
    dst = edges_index[1].astype(jnp.int32)
    src = jnp.concatenate([src, jnp.zeros((EPAD - E,), jnp.int32)])
    dst = jnp.concatenate([dst, jnp.full((EPAD - E,), N, jnp.int32)])

    degp = _sc_degree(dst)
    h1 = _tc_matmul(X, W1)
    g1, dv = _tc_scale(h1, degp[0, :N], degp[1, :N])

    p = _sc_aggregate(g1, src, dst)
    h2, g2 = _tc_combine_matmul(p[0, :N], p[1, :N], h1, dv, b1, W2)
    p = _sc_aggregate(g2, src, dst)
    h3, g3 = _tc_combine_matmul(p[0, :N], p[1, :N], h2, dv, b2, W3)
    p = _sc_aggregate(g3, src, dst)
    return _tc_head(p[0, :N], p[1, :N], h3, dv, b3, Wout, bout)
